# baseline (speedup 1.0000x reference)
import functools

import jax
import jax.numpy as jnp
from jax import lax
from jax.experimental import pallas as pl
from jax.experimental.pallas import tpu as pltpu
from jax.experimental.pallas import tpu_sc as plsc

BATCH = 16384
DIM = 64
LANES = 16

_info = plsc.get_sparse_core_info()
_NC, _NS = _info.num_cores, _info.num_subcores
_NW = _NC * _NS
_BPW = BATCH // _NW
_CHUNK = 128
_PACK_BLK = 16384
_LOG2_PB = _PACK_BLK.bit_length() - 1


def _body(lat_hbm, label_hbm, tab_hbm, out_hbm, idx_v, pidx_v, off_v,
          lat_v, pairs_a, pairs_b, out_v, sem_a, sem_b):
    wid = lax.axis_index("s") * _NC + lax.axis_index("c")
    base = wid * _BPW

    pltpu.sync_copy(label_hbm.at[pl.ds(base, _BPW)], idx_v)
    pltpu.sync_copy(lat_hbm.at[:, pl.ds(base, _BPW)], lat_v)

    def prep(g, carry):
        sl = pl.ds(g * LANES, LANES)
        v = idx_v[sl]
        pidx_v[sl] = lax.bitwise_or(
            lax.shift_left(lax.shift_right_logical(v, _LOG2_PB + 1), _LOG2_PB),
            lax.bitwise_and(v, _PACK_BLK - 1),
        )
        off_v[sl] = lax.shift_left(
            lax.bitwise_and(lax.shift_right_logical(v, _LOG2_PB), 1), 6
        )
        return carry

    lax.fori_loop(0, _BPW // LANES, prep, 0)

    lane = lax.iota(jnp.int32, LANES)

    bufs = (pairs_a, pairs_b)
    sems = (sem_a, sem_b)

    def fire(c):
        pltpu.async_copy(
            tab_hbm.at[pidx_v.at[pl.ds(c * _CHUNK, _CHUNK)]],
            bufs[c % 2],
            sems[c % 2],
        )

    for c in range(_BPW // _CHUNK):
        fire(c)
        pltpu.make_async_copy(
            tab_hbm.at[pidx_v.at[pl.ds(c * _CHUNK, _CHUNK)]],
            bufs[c % 2],
            sems[c % 2],
        ).wait()
        buf = bufs[c % 2]

        def sel_mul(g, carry):
            sl = pl.ds(c * _CHUNK + g * LANES, LANES)
            i0 = lane + g * LANES
            off = off_v[sl]

            def col(j, cc):
                vals = plsc.load_gather(buf, [i0, off + j])
                out_v[j, sl] = vals * lat_v[j, sl]
                return cc

            lax.fori_loop(0, DIM, col, 0)
            return carry

        lax.fori_loop(0, _CHUNK // LANES, sel_mul, 0)

    pltpu.sync_copy(out_v, out_hbm.at[:, pl.ds(base, _BPW)])


def _pack_body(tab_t_ref, out_ref):
    x = tab_t_ref[...]
    lo = x[:, :_PACK_BLK].T
    hi = x[:, _PACK_BLK:].T
    out_ref[...] = jnp.concatenate([lo, hi], axis=1)


def _pack(tab_t):
    grid = (tab_t.shape[1] + 2 * _PACK_BLK - 1) // (2 * _PACK_BLK)
    return pl.pallas_call(
        _pack_body,
        grid=(grid,),
        in_specs=[pl.BlockSpec((DIM, 2 * _PACK_BLK), lambda p: (0, p))],
        out_specs=pl.BlockSpec((_PACK_BLK, 2 * DIM), lambda p: (p, 0)),
        out_shape=jax.ShapeDtypeStruct((grid * _PACK_BLK, 2 * DIM), jnp.float32),
        compiler_params=pltpu.CompilerParams(vmem_limit_bytes=128 * 1024 * 1024),
    )(tab_t)


@jax.jit
def _run(lat_t, label, tab2):
    mesh = plsc.VectorSubcoreMesh(core_axis_name="c", subcore_axis_name="s")
    kern = functools.partial(
        pl.kernel,
        mesh=mesh,
        out_type=jax.ShapeDtypeStruct((DIM, BATCH), jnp.float32),
        scratch_types=[
            pltpu.VMEM((_BPW,), jnp.int32),
            pltpu.VMEM((_BPW,), jnp.int32),
            pltpu.VMEM((_BPW,), jnp.int32),
            pltpu.VMEM((DIM, _BPW), jnp.float32),
            pltpu.VMEM((_CHUNK, 2 * DIM), jnp.float32),
            pltpu.VMEM((_CHUNK, 2 * DIM), jnp.float32),
            pltpu.VMEM((DIM, _BPW), jnp.float32),
            pltpu.SemaphoreType.DMA,
            pltpu.SemaphoreType.DMA,
        ],
        compiler_params=pltpu.CompilerParams(needs_layout_passes=False),
    )(_body)
    return kern(lat_t, label, tab2)


def kernel(latent, label, table):
    tab2 = _pack(table.T)
    out_t = _run(latent.T, label.astype(jnp.int32), tab2)
    return out_t.T


# speedup vs baseline: 1.0104x; 1.0104x over previous
import functools

import jax
import jax.numpy as jnp
from jax import lax
from jax.experimental import pallas as pl
from jax.experimental.pallas import tpu as pltpu
from jax.experimental.pallas import tpu_sc as plsc

BATCH = 16384
DIM = 64
LANES = 16

_info = plsc.get_sparse_core_info()
_NC, _NS = _info.num_cores, _info.num_subcores
_NW = _NC * _NS
_BPW = BATCH // _NW
_CHUNK = 128
_PACK_BLK = 16384
_LOG2_PB = _PACK_BLK.bit_length() - 1


def _body(lat_hbm, label_hbm, tab_hbm, out_hbm, idx_v, pidx_v, off_v,
          lat_v, pairs_a, pairs_b, out_v, sem_a, sem_b):
    wid = lax.axis_index("s") * _NC + lax.axis_index("c")
    base = wid * _BPW

    pltpu.sync_copy(label_hbm.at[pl.ds(base, _BPW)], idx_v)
    pltpu.sync_copy(lat_hbm.at[:, pl.ds(base, _BPW)], lat_v)

    def prep(g, carry):
        sl = pl.ds(g * LANES, LANES)
        v = idx_v[sl]
        pidx_v[sl] = lax.bitwise_or(
            lax.shift_left(lax.shift_right_logical(v, _LOG2_PB + 1), _LOG2_PB),
            lax.bitwise_and(v, _PACK_BLK - 1),
        )
        off_v[sl] = lax.shift_left(
            lax.bitwise_and(lax.shift_right_logical(v, _LOG2_PB), 1), 6
        )
        return carry

    lax.fori_loop(0, _BPW // LANES, prep, 0)

    lane = lax.iota(jnp.int32, LANES)

    bufs = (pairs_a, pairs_b)
    sems = (sem_a, sem_b)

    def fire(c):
        pltpu.async_copy(
            tab_hbm.at[pidx_v.at[pl.ds(c * _CHUNK, _CHUNK)]],
            bufs[c % 2],
            sems[c % 2],
        )

    fire(0)
    for c in range(_BPW // _CHUNK):
        if c + 1 < _BPW // _CHUNK:
            fire(c + 1)
        pltpu.make_async_copy(
            tab_hbm.at[pidx_v.at[pl.ds(c * _CHUNK, _CHUNK)]],
            bufs[c % 2],
            sems[c % 2],
        ).wait()
        buf = bufs[c % 2]

        def sel_mul(g, carry):
            sl = pl.ds(c * _CHUNK + g * LANES, LANES)
            i0 = lane + g * LANES
            off = off_v[sl]

            def col(j, cc):
                vals = plsc.load_gather(buf, [i0, off + j])
                out_v[j, sl] = vals * lat_v[j, sl]
                return cc

            lax.fori_loop(0, DIM, col, 0, unroll=8)
            return carry

        lax.fori_loop(0, _CHUNK // LANES, sel_mul, 0)

    pltpu.sync_copy(out_v, out_hbm.at[:, pl.ds(base, _BPW)])


def _pack_body(tab_t_ref, out_ref):
    x = tab_t_ref[...]
    lo = x[:, :_PACK_BLK].T
    hi = x[:, _PACK_BLK:].T
    out_ref[...] = jnp.concatenate([lo, hi], axis=1)


def _pack(tab_t):
    grid = (tab_t.shape[1] + 2 * _PACK_BLK - 1) // (2 * _PACK_BLK)
    return pl.pallas_call(
        _pack_body,
        grid=(grid,),
        in_specs=[pl.BlockSpec((DIM, 2 * _PACK_BLK), lambda p: (0, p))],
        out_specs=pl.BlockSpec((_PACK_BLK, 2 * DIM), lambda p: (p, 0)),
        out_shape=jax.ShapeDtypeStruct((grid * _PACK_BLK, 2 * DIM), jnp.float32),
        compiler_params=pltpu.CompilerParams(vmem_limit_bytes=128 * 1024 * 1024),
    )(tab_t)


@jax.jit
def _run(lat_t, label, tab2):
    mesh = plsc.VectorSubcoreMesh(core_axis_name="c", subcore_axis_name="s")
    kern = functools.partial(
        pl.kernel,
        mesh=mesh,
        out_type=jax.ShapeDtypeStruct((DIM, BATCH), jnp.float32),
        scratch_types=[
            pltpu.VMEM((_BPW,), jnp.int32),
            pltpu.VMEM((_BPW,), jnp.int32),
            pltpu.VMEM((_BPW,), jnp.int32),
            pltpu.VMEM((DIM, _BPW), jnp.float32),
            pltpu.VMEM((_CHUNK, 2 * DIM), jnp.float32),
            pltpu.VMEM((_CHUNK, 2 * DIM), jnp.float32),
            pltpu.VMEM((DIM, _BPW), jnp.float32),
            pltpu.SemaphoreType.DMA,
            pltpu.SemaphoreType.DMA,
        ],
        compiler_params=pltpu.CompilerParams(needs_layout_passes=False),
    )(_body)
    return kern(lat_t, label, tab2)


def kernel(latent, label, table):
    tab2 = _pack(table.T)
    out_t = _run(latent.T, label.astype(jnp.int32), tab2)
    return out_t.T


# speedup vs baseline: 1.4165x; 1.4020x over previous
import functools

import jax
import jax.numpy as jnp
from jax import lax
from jax.experimental import pallas as pl
from jax.experimental.pallas import tpu as pltpu
from jax.experimental.pallas import tpu_sc as plsc

BATCH = 16384
DIM = 64
LANES = 16

_info = plsc.get_sparse_core_info()
_NC, _NS = _info.num_cores, _info.num_subcores
_NW = _NC * _NS
_BPW = BATCH // _NW
_CHUNK = 128
_Q = 8192
_LOG2_Q = _Q.bit_length() - 1


def _body(lat_hbm, label_hbm, tab_hbm, out_hbm, idx_v, pidx_v, off_v, sel_v,
          lat_v, pairs_a, pairs_b, out_v, sem_a, sem_b):
    wid = lax.axis_index("s") * _NC + lax.axis_index("c")
    base = wid * _BPW

    pltpu.sync_copy(label_hbm.at[pl.ds(base, _BPW)], idx_v)
    pltpu.sync_copy(lat_hbm.at[:, pl.ds(base, _BPW)], lat_v)

    def prep(g, carry):
        sl = pl.ds(g * LANES, LANES)
        v = idx_v[sl]
        pidx_v[sl] = lax.bitwise_or(
            lax.shift_left(lax.shift_right_logical(v, _LOG2_Q + 2), _LOG2_Q),
            lax.bitwise_and(v, _Q - 1),
        )
        q = lax.bitwise_and(lax.shift_right_logical(v, _LOG2_Q), 3)
        off_v[sl] = lax.shift_left(lax.bitwise_and(q, 1), 6)
        sel_v[sl] = lax.shift_right_logical(q, 1)
        return carry

    lax.fori_loop(0, _BPW // LANES, prep, 0)

    lane = lax.iota(jnp.int32, LANES)

    bufs = (pairs_a, pairs_b)
    sems = (sem_a, sem_b)

    def fire(c):
        pltpu.async_copy(
            tab_hbm.at[pidx_v.at[pl.ds(c * _CHUNK, _CHUNK)]],
            bufs[c % 2],
            sems[c % 2],
        )

    fire(0)
    for c in range(_BPW // _CHUNK):
        if c + 1 < _BPW // _CHUNK:
            fire(c + 1)
        pltpu.make_async_copy(
            tab_hbm.at[pidx_v.at[pl.ds(c * _CHUNK, _CHUNK)]],
            bufs[c % 2],
            sems[c % 2],
        ).wait()
        buf = bufs[c % 2]

        def sel_mul(g, carry):
            sl = pl.ds(c * _CHUNK + g * LANES, LANES)
            i0 = lane + g * LANES
            off = off_v[sl]
            keep_hi = sel_v[sl] == 0
            mask_hi = jnp.full((LANES,), 0xFFFF0000, jnp.uint32)

            def col(j, cc):
                vals = plsc.load_gather(buf, [i0, off + j])
                bits = plsc.bitcast(vals, jnp.uint32)
                picked = jnp.where(
                    keep_hi,
                    lax.bitwise_and(bits, mask_hi),
                    lax.shift_left(bits, jnp.uint32(16)),
                )
                out_v[j, sl] = plsc.bitcast(picked, jnp.float32) * lat_v[j, sl]
                return cc

            lax.fori_loop(0, DIM, col, 0, unroll=8)
            return carry

        lax.fori_loop(0, _CHUNK // LANES, sel_mul, 0)

    pltpu.sync_copy(out_v, out_hbm.at[:, pl.ds(base, _BPW)])


def _bf16_word(a, b):
    ua = lax.bitcast_convert_type(a, jnp.uint32)
    ub = lax.bitcast_convert_type(b, jnp.uint32)
    ra = lax.bitwise_and(ua + jnp.uint32(0x8000), jnp.uint32(0xFFFF0000))
    rb = lax.shift_right_logical(ub + jnp.uint32(0x8000), jnp.uint32(16))
    return lax.bitcast_convert_type(lax.bitwise_or(ra, rb), jnp.float32)


def _pack_body(tab_t_ref, out_ref):
    x = tab_t_ref[...]
    w0 = _bf16_word(x[:, 0 * _Q:1 * _Q], x[:, 2 * _Q:3 * _Q]).T
    w1 = _bf16_word(x[:, 1 * _Q:2 * _Q], x[:, 3 * _Q:4 * _Q]).T
    out_ref[...] = jnp.concatenate([w0, w1], axis=1)


def _pack(tab_t):
    grid = (tab_t.shape[1] + 4 * _Q - 1) // (4 * _Q)
    return pl.pallas_call(
        _pack_body,
        grid=(grid,),
        in_specs=[pl.BlockSpec((DIM, 4 * _Q), lambda p: (0, p))],
        out_specs=pl.BlockSpec((_Q, 2 * DIM), lambda p: (p, 0)),
        out_shape=jax.ShapeDtypeStruct((grid * _Q, 2 * DIM), jnp.float32),
        compiler_params=pltpu.CompilerParams(
            vmem_limit_bytes=128 * 1024 * 1024,
        ),
    )(tab_t)


@jax.jit
def _run(lat_t, label, tab2):
    mesh = plsc.VectorSubcoreMesh(core_axis_name="c", subcore_axis_name="s")
    kern = functools.partial(
        pl.kernel,
        mesh=mesh,
        out_type=jax.ShapeDtypeStruct((DIM, BATCH), jnp.float32),
        scratch_types=[
            pltpu.VMEM((_BPW,), jnp.int32),
            pltpu.VMEM((_BPW,), jnp.int32),
            pltpu.VMEM((_BPW,), jnp.int32),
            pltpu.VMEM((_BPW,), jnp.int32),
            pltpu.VMEM((DIM, _BPW), jnp.float32),
            pltpu.VMEM((_CHUNK, 2 * DIM), jnp.float32),
            pltpu.VMEM((_CHUNK, 2 * DIM), jnp.float32),
            pltpu.VMEM((DIM, _BPW), jnp.float32),
            pltpu.SemaphoreType.DMA,
            pltpu.SemaphoreType.DMA,
        ],
        compiler_params=pltpu.CompilerParams(needs_layout_passes=False),
    )(_body)
    return kern(lat_t, label, tab2)


def kernel(latent, label, table):
    tab2 = _pack(table.T)
    out_t = _run(latent.T, label.astype(jnp.int32), tab2)
    return out_t.T
